# baseline (speedup 1.0000x reference)
import functools

import jax
import jax.numpy as jnp
from jax.experimental import pallas as pl
from jax.experimental.pallas import tpu as pltpu

_DIM = 2048
_KEEP_RANK = _DIM // 2
_EPS = 1e-08
_CHUNK = 256


def _body(mu_row_ref, logD_row_ref, mu_col_ref, logD_col_ref, x_ref, o_ref,
          mask_ref):
    @pl.when(pl.program_id(0) == 0)
    def _compute_mask():
        la_row = logD_row_ref[...] - jnp.log(mu_row_ref[...] ** 2 + _EPS)
        la_col = logD_col_ref[...] - jnp.log(mu_col_ref[...] ** 2 + _EPS)
        counts = jnp.zeros((1, _DIM), dtype=jnp.int32)
        for k in range(_DIM // _CHUNK):
            la_i = la_col[k * _CHUNK:(k + 1) * _CHUNK, :]
            counts = counts + jnp.sum(
                (la_i < la_row).astype(jnp.int32), axis=0, keepdims=True)
        mask_ref[...] = (counts >= _KEEP_RANK).astype(jnp.float32)
        n_below = jnp.sum((counts < _KEEP_RANK).astype(jnp.int32))

        @pl.when(n_below != _KEEP_RANK)
        def _tie_fix():
            j_idx = jax.lax.broadcasted_iota(jnp.int32, (_CHUNK, _DIM), 1)
            eqc = jnp.zeros((1, _DIM), dtype=jnp.int32)
            for k in range(_DIM // _CHUNK):
                la_i = la_col[k * _CHUNK:(k + 1) * _CHUNK, :]
                i_idx = k * _CHUNK + jax.lax.broadcasted_iota(
                    jnp.int32, (_CHUNK, _DIM), 0)
                eq_before = (la_i == la_row) & (i_idx < j_idx)
                eqc = eqc + jnp.sum(
                    eq_before.astype(jnp.int32), axis=0, keepdims=True)
            mask_ref[...] = ((counts + eqc) >= _KEEP_RANK).astype(jnp.float32)

    o_ref[...] = x_ref[...] * mask_ref[...]


@functools.partial(jax.jit, static_argnames=("block_rows",))
def _run(x, mu, logD, block_rows=1024):
    mu_row = mu.reshape(1, _DIM)
    logD_row = logD.reshape(1, _DIM)
    mu_col = mu.reshape(_DIM, 1)
    logD_col = logD.reshape(_DIM, 1)

    rows = x.shape[0] * x.shape[1]
    x2d = x.reshape(rows, _DIM)
    y2d = pl.pallas_call(
        _body,
        grid=(rows // block_rows,),
        in_specs=[
            pl.BlockSpec((1, _DIM), lambda i: (0, 0)),
            pl.BlockSpec((1, _DIM), lambda i: (0, 0)),
            pl.BlockSpec((_DIM, 1), lambda i: (0, 0)),
            pl.BlockSpec((_DIM, 1), lambda i: (0, 0)),
            pl.BlockSpec((block_rows, _DIM), lambda i: (i, 0)),
        ],
        out_specs=pl.BlockSpec((block_rows, _DIM), lambda i: (i, 0)),
        out_shape=jax.ShapeDtypeStruct((rows, _DIM), jnp.float32),
        scratch_shapes=[pltpu.VMEM((1, _DIM), jnp.float32)],
        compiler_params=pltpu.CompilerParams(
            dimension_semantics=("arbitrary",)),
    )(mu_row, logD_row, mu_col, logD_col, x2d)
    return y2d.reshape(x.shape)


def kernel(x, mu, logD):
    return _run(x, mu, logD)


# speedup vs baseline: 1.0322x; 1.0322x over previous
import functools

import jax
import jax.numpy as jnp
from jax.experimental import pallas as pl
from jax.experimental.pallas import tpu as pltpu

_DIM = 2048
_KEEP_RANK = _DIM // 2
_EPS = 1e-08
_CHUNK = 256


def _body(mu_row_ref, logD_row_ref, x_ref, o_ref, mask_ref):
    @pl.when(pl.program_id(0) == 0)
    def _compute_mask():
        la_row = logD_row_ref[...] - jnp.log(mu_row_ref[...] ** 2 + _EPS)
        la_col = la_row.reshape(_DIM, 1)
        counts = jnp.zeros((1, _DIM), dtype=jnp.int32)
        for k in range(_DIM // _CHUNK):
            la_i = la_col[k * _CHUNK:(k + 1) * _CHUNK, :]
            counts = counts + jnp.sum(
                (la_i < la_row).astype(jnp.int32), axis=0, keepdims=True)
        mask_ref[...] = (counts >= _KEEP_RANK).astype(jnp.float32)
        n_below = jnp.sum((counts < _KEEP_RANK).astype(jnp.int32))

        @pl.when(n_below != _KEEP_RANK)
        def _tie_fix():
            j_idx = jax.lax.broadcasted_iota(jnp.int32, (_CHUNK, _DIM), 1)
            eqc = jnp.zeros((1, _DIM), dtype=jnp.int32)
            for k in range(_DIM // _CHUNK):
                la_i = la_col[k * _CHUNK:(k + 1) * _CHUNK, :]
                i_idx = k * _CHUNK + jax.lax.broadcasted_iota(
                    jnp.int32, (_CHUNK, _DIM), 0)
                eq_before = (la_i == la_row) & (i_idx < j_idx)
                eqc = eqc + jnp.sum(
                    eq_before.astype(jnp.int32), axis=0, keepdims=True)
            mask_ref[...] = ((counts + eqc) >= _KEEP_RANK).astype(jnp.float32)

    o_ref[...] = x_ref[...] * mask_ref[...]


@functools.partial(jax.jit, static_argnames=("block_rows",))
def _run(x, mu, logD, block_rows=1024):
    mu_row = mu.reshape(1, _DIM)
    logD_row = logD.reshape(1, _DIM)

    rows = x.shape[0] * x.shape[1]
    x2d = x.reshape(rows, _DIM)
    y2d = pl.pallas_call(
        _body,
        grid=(rows // block_rows,),
        in_specs=[
            pl.BlockSpec((1, _DIM), lambda i: (0, 0)),
            pl.BlockSpec((1, _DIM), lambda i: (0, 0)),
            pl.BlockSpec((block_rows, _DIM), lambda i: (i, 0)),
        ],
        out_specs=pl.BlockSpec((block_rows, _DIM), lambda i: (i, 0)),
        out_shape=jax.ShapeDtypeStruct((rows, _DIM), jnp.float32),
        scratch_shapes=[pltpu.VMEM((1, _DIM), jnp.float32)],
        compiler_params=pltpu.CompilerParams(
            dimension_semantics=("arbitrary",)),
    )(mu_row, logD_row, x2d)
    return y2d.reshape(x.shape)


def kernel(x, mu, logD):
    return _run(x, mu, logD)
